# baseline (device time: 566144 ns/iter reference)
import jax
import jax.numpy as jnp
from jax import lax
from jax.experimental import pallas as pl
from jax.experimental.pallas import tpu as pltpu

N_Z = 4
PAGES_PER_SHARD = 64
BS = 16
H = 8
D = 64
B = 8
ROWS = PAGES_PER_SHARD * BS
COLS = H * D
NC = B * H
SROWS = NC + 2


def _body(k_ref, v_ref, qbd_ref, w_ref, out_ref, comm_ref, send_sems, recv_sems):
    me = lax.axis_index("z")
    mx = lax.axis_index("x")
    my = lax.axis_index("y")
    left = (me + N_Z - 1) % N_Z
    right = (me + 1) % N_Z

    barrier_sem = pltpu.get_barrier_semaphore()
    for nbr in [left, right]:
        pl.semaphore_signal(
            barrier_sem, inc=1,
            device_id=(mx, my, nbr),
            device_id_type=pl.DeviceIdType.MESH,
        )
    pl.semaphore_wait(barrier_sem, 2)

    S = jnp.dot(k_ref[...], qbd_ref[...], preferred_element_type=jnp.float32)
    S = S * (D ** -0.5)
    W = w_ref[...]
    Sm = jnp.where(W > 0, S, -1e30)
    m = jnp.max(Sm, axis=0)
    E = W * jnp.exp(Sm - m[None, :])
    l = jnp.sum(E, axis=0)
    O = lax.dot_general(
        E.astype(jnp.bfloat16), v_ref[...],
        dimension_numbers=(((0,), (0,)), ((), ())),
        preferred_element_type=jnp.float32,
    )

    comm_ref[pl.ds(me, 1), 0:NC, :] = O[None]
    comm_ref[pl.ds(me, 1), NC:NC + 1, 0:NC] = m.reshape(1, 1, NC)
    comm_ref[pl.ds(me, 1), NC + 1:NC + 2, 0:NC] = l.reshape(1, 1, NC)

    for h in range(N_Z - 1):
        o = (me + N_Z - h) % N_Z
        rdma = pltpu.make_async_remote_copy(
            src_ref=comm_ref.at[pl.ds(o, 1)],
            dst_ref=comm_ref.at[pl.ds(o, 1)],
            send_sem=send_sems.at[h],
            recv_sem=recv_sems.at[h],
            device_id=(mx, my, right),
            device_id_type=pl.DeviceIdType.MESH,
        )
        rdma.start()
        rdma.wait()

    ms = [comm_ref[p, NC, 0:NC] for p in range(N_Z)]
    Mx = jnp.maximum(jnp.maximum(ms[0], ms[1]), jnp.maximum(ms[2], ms[3]))
    sc = [jnp.exp(ms[p] - Mx) for p in range(N_Z)]
    L = sum(sc[p] * comm_ref[p, NC + 1, 0:NC] for p in range(N_Z))
    Ofin = sum(comm_ref[p, 0:NC, :] * sc[p][:, None] for p in range(N_Z))
    out_ref[...] = Ofin / L[:, None]


def kernel(Q, K, V, bt, lens):
    bf16 = jnp.bfloat16
    Kp = K.reshape(ROWS, COLS).astype(bf16)
    Vp = V.reshape(ROWS, COLS).astype(bf16)

    Qbd = jnp.zeros((COLS, NC), bf16)
    for h in range(H):
        Qbd = Qbd.at[h * D:(h + 1) * D, h::H].set(Q[:, 0, h, :].T.astype(bf16))

    z = lax.axis_index("z")
    pages_local = z * PAGES_PER_SHARD + jnp.arange(PAGES_PER_SHARD)
    valid = jnp.arange(64)[None, :] < lens[:, None]
    cnt = jnp.sum(
        (bt[:, :, None] == pages_local[None, None, :]) & valid[:, :, None],
        axis=1,
    ).astype(jnp.float32)
    Wrow = jnp.repeat(cnt, BS, axis=1)
    W = jnp.repeat(Wrow.T, H, axis=1)

    Ofin = pl.pallas_call(
        _body,
        out_shape=jax.ShapeDtypeStruct((NC, COLS), jnp.float32),
        in_specs=[pl.BlockSpec(memory_space=pltpu.VMEM)] * 4,
        out_specs=pl.BlockSpec(memory_space=pltpu.VMEM),
        scratch_shapes=[
            pltpu.VMEM((N_Z, SROWS, COLS), jnp.float32),
            pltpu.SemaphoreType.DMA((N_Z - 1,)),
            pltpu.SemaphoreType.DMA((N_Z - 1,)),
        ],
        compiler_params=pltpu.CompilerParams(collective_id=0),
    )(Kp, Vp, Qbd, W)

    o_bh = Ofin.reshape(B, H, COLS)
    out = jnp.stack([o_bh[:, h, h * D:(h + 1) * D] for h in range(H)], axis=1)
    return out.reshape(B, 1, H, D)


# device time: 22608 ns/iter; 25.0418x vs baseline; 25.0418x over previous
import jax
import jax.numpy as jnp
from jax import lax
from jax.experimental import pallas as pl
from jax.experimental.pallas import tpu as pltpu

N_Z = 4
PAGES_PER_SHARD = 64
BS = 16
H = 8
D = 64
B = 8
ROWS = PAGES_PER_SHARD * BS
COLS = H * D
NC = B * H
SROWS = NC + 2


def _body(k_ref, v_ref, qbd_ref, w_ref, out_ref, comm_ref, send_sems, recv_sems):
    me = lax.axis_index("z")
    mx = lax.axis_index("x")
    my = lax.axis_index("y")
    left = (me + N_Z - 1) % N_Z
    right = (me + 1) % N_Z

    barrier_sem = pltpu.get_barrier_semaphore()
    for nbr in [left, right]:
        pl.semaphore_signal(
            barrier_sem, inc=1,
            device_id=(mx, my, nbr),
            device_id_type=pl.DeviceIdType.MESH,
        )
    pl.semaphore_wait(barrier_sem, 2)

    S = jnp.dot(k_ref[...], qbd_ref[...], preferred_element_type=jnp.float32)
    S = S * (D ** -0.5)
    W = w_ref[...]
    Sm = jnp.where(W > 0, S, -1e30)
    m = jnp.max(Sm, axis=0)
    E = W * jnp.exp(Sm - m[None, :])
    l = jnp.sum(E, axis=0)
    O = lax.dot_general(
        E.astype(jnp.bfloat16), v_ref[...],
        dimension_numbers=(((0,), (0,)), ((), ())),
        preferred_element_type=jnp.float32,
    )

    comm_ref[pl.ds(me, 1), 0:NC, :] = O[None]
    comm_ref[pl.ds(me, 1), NC:NC + 1, 0:NC] = m.reshape(1, 1, NC)
    comm_ref[pl.ds(me, 1), NC + 1:NC + 2, 0:NC] = l.reshape(1, 1, NC)

    for h in range(N_Z - 1):
        o = (me + N_Z - h) % N_Z
        rdma = pltpu.make_async_remote_copy(
            src_ref=comm_ref.at[pl.ds(o, 1)],
            dst_ref=comm_ref.at[pl.ds(o, 1)],
            send_sem=send_sems.at[h],
            recv_sem=recv_sems.at[h],
            device_id=(mx, my, right),
            device_id_type=pl.DeviceIdType.MESH,
        )
        rdma.start()
        rdma.wait()

    ms = [comm_ref[p, NC, 0:NC] for p in range(N_Z)]
    Mx = jnp.maximum(jnp.maximum(ms[0], ms[1]), jnp.maximum(ms[2], ms[3]))
    sc = [jnp.exp(ms[p] - Mx) for p in range(N_Z)]
    L = sum(sc[p] * comm_ref[p, NC + 1, 0:NC] for p in range(N_Z))
    Ofin = sum(comm_ref[p, 0:NC, :] * sc[p][:, None] for p in range(N_Z))
    out_ref[...] = Ofin / L[:, None]


def kernel(Q, K, V, bt, lens):
    bf16 = jnp.bfloat16
    Kp = K.reshape(ROWS, COLS).astype(bf16)
    Vp = V.reshape(ROWS, COLS).astype(bf16)

    I8 = jnp.eye(H, dtype=bf16)
    Q2 = Q[:, 0, :, :].astype(bf16)
    Qbd = jnp.einsum("bhd,hg->hdbg", Q2, I8).reshape(COLS, NC)

    z = lax.axis_index("z")
    pages_local = z * PAGES_PER_SHARD + jnp.arange(PAGES_PER_SHARD)
    valid = jnp.arange(64)[None, :] < lens[:, None]
    cnt = jnp.sum(
        (bt[:, :, None] == pages_local[None, None, :]) & valid[:, :, None],
        axis=1,
    ).astype(jnp.float32)
    Wrow = jnp.repeat(cnt, BS, axis=1)
    W = jnp.repeat(Wrow.T, H, axis=1)

    Ofin = pl.pallas_call(
        _body,
        out_shape=jax.ShapeDtypeStruct((NC, COLS), jnp.float32),
        in_specs=[pl.BlockSpec(memory_space=pltpu.VMEM)] * 4,
        out_specs=pl.BlockSpec(memory_space=pltpu.VMEM),
        scratch_shapes=[
            pltpu.VMEM((N_Z, SROWS, COLS), jnp.float32),
            pltpu.SemaphoreType.DMA((N_Z - 1,)),
            pltpu.SemaphoreType.DMA((N_Z - 1,)),
        ],
        compiler_params=pltpu.CompilerParams(collective_id=0),
    )(Kp, Vp, Qbd, W)

    R = Ofin.reshape(B, H, H, D)
    out = jnp.einsum("bghd,gh->bhd", R, jnp.eye(H, dtype=Ofin.dtype))
    return out.reshape(B, 1, H, D)
